# Initial kernel scaffold; baseline (speedup 1.0000x reference)
import jax
import jax.numpy as jnp
from jax.experimental import pallas as pl


def kernel(L_fwd, L_bwd, W_fwd, W_bwd):
    raise NotImplementedError("write your pallas kernel here")


# speedup vs baseline: 3.6471x; 3.6471x over previous
import functools

import jax
import jax.numpy as jnp
from jax import lax
from jax.experimental import pallas as pl
from jax.experimental.pallas import tpu as pltpu
from jax.experimental.pallas import tpu_sc as plsc

NC = 2
NS = 16
LANES = 16
NW = NC * NS

ROWS = 1024 * 1000
D = 128
PER_W = ROWS // NW
CHUNK = 128
NCHUNK = PER_W // CHUNK

_mesh = plsc.VectorSubcoreMesh(core_axis_name="c", subcore_axis_name="s")


@functools.partial(
    pl.kernel,
    out_type=jax.ShapeDtypeStruct((ROWS, D), jnp.float32),
    mesh=_mesh,
    scratch_types=[
        pltpu.VMEM((CHUNK,), jnp.int32),
        pltpu.VMEM((CHUNK,), jnp.int32),
        pltpu.VMEM((CHUNK, D), jnp.float32),
        pltpu.VMEM((CHUNK, D), jnp.float32),
        pltpu.SemaphoreType.DMA,
        pltpu.SemaphoreType.DMA,
    ],
)
def _encode(lf_hbm, lb_hbm, wf_hbm, wb_hbm, out_hbm,
            idx_f, idx_b, buf_f, buf_b, sem_f, sem_b):
    wid = lax.axis_index("s") * NC + lax.axis_index("c")
    base = wid * PER_W

    def chunk_body(c, carry):
        off = base + c * CHUNK
        pltpu.sync_copy(lf_hbm.at[pl.ds(off, CHUNK)], idx_f)
        pltpu.sync_copy(lb_hbm.at[pl.ds(off, CHUNK)], idx_b)
        cf = pltpu.async_copy(wf_hbm.at[idx_f], buf_f, sem_f)
        cb = pltpu.async_copy(wb_hbm.at[idx_b], buf_b, sem_b)
        cf.wait()
        cb.wait()

        def add_body(i, carry2):
            for j in range(D // LANES):
                s = pl.ds(j * LANES, LANES)
                buf_f[i, s] = buf_f[i, s] + buf_b[i, s]
            return carry2

        lax.fori_loop(0, CHUNK, add_body, 0, unroll=2)
        pltpu.sync_copy(buf_f, out_hbm.at[pl.ds(off, CHUNK)])
        return carry

    lax.fori_loop(0, NCHUNK, chunk_body, 0)


def kernel(L_fwd, L_bwd, W_fwd, W_bwd):
    out = _encode(L_fwd.reshape(-1), L_bwd.reshape(-1), W_fwd, W_bwd)
    return out.reshape(L_fwd.shape[0], L_fwd.shape[1], D)


# speedup vs baseline: 5.6126x; 1.5389x over previous
import functools

import jax
import jax.numpy as jnp
from jax import lax
from jax.experimental import pallas as pl
from jax.experimental.pallas import tpu as pltpu
from jax.experimental.pallas import tpu_sc as plsc

NC = 2
NS = 16
LANES = 16
NW = NC * NS

ROWS = 1024 * 1000
D = 128
PER_W = ROWS // NW
CHUNK = 64
NCHUNK = PER_W // CHUNK

_mesh = plsc.VectorSubcoreMesh(core_axis_name="c", subcore_axis_name="s")


@functools.partial(
    pl.kernel,
    out_type=jax.ShapeDtypeStruct((ROWS, D), jnp.float32),
    mesh=_mesh,
    scratch_types=[
        pltpu.VMEM((4, CHUNK), jnp.int32),
        pltpu.VMEM((4, CHUNK), jnp.int32),
        pltpu.VMEM((2, CHUNK, D), jnp.float32),
        pltpu.VMEM((2, CHUNK, D), jnp.float32),
        pltpu.VMEM((2, CHUNK, D), jnp.float32),
        pltpu.SemaphoreType.DMA,
        pltpu.SemaphoreType.DMA,
        pltpu.SemaphoreType.DMA,
        pltpu.SemaphoreType.DMA,
        pltpu.SemaphoreType.DMA,
        pltpu.SemaphoreType.DMA,
        pltpu.SemaphoreType.DMA,
        pltpu.SemaphoreType.DMA,
    ],
)
def _encode(lf, lb, wf, wb, out,
            idx_f, idx_b, buf_f, buf_b, buf_s,
            si0, si1, si2, si3, sg0, sg1, so0, so1):
    si = (si0, si1, si2, si3)
    sg = (sg0, sg1)
    so = (so0, so1)
    wid = lax.axis_index("s") * NC + lax.axis_index("c")
    base = wid * PER_W

    def idx_start(c, sl):
        off = base + c * CHUNK
        pltpu.async_copy(lf.at[pl.ds(off, CHUNK)], idx_f.at[sl], si[sl])
        pltpu.async_copy(lb.at[pl.ds(off, CHUNK)], idx_b.at[sl], si[sl])

    def idx_wait(c, sl):
        off = base + c * CHUNK
        pltpu.make_async_copy(lf.at[pl.ds(off, CHUNK)], idx_f.at[sl], si[sl]).wait()
        pltpu.make_async_copy(lb.at[pl.ds(off, CHUNK)], idx_b.at[sl], si[sl]).wait()

    def gather_start(i4, g2):
        pltpu.async_copy(wf.at[idx_f.at[i4]], buf_f.at[g2], sg[g2])
        pltpu.async_copy(wb.at[idx_b.at[i4]], buf_b.at[g2], sg[g2])

    def gather_wait(i4, g2):
        pltpu.make_async_copy(wf.at[idx_f.at[i4]], buf_f.at[g2], sg[g2]).wait()
        pltpu.make_async_copy(wb.at[idx_b.at[i4]], buf_b.at[g2], sg[g2]).wait()

    def out_start(c, g2):
        off = base + c * CHUNK
        pltpu.async_copy(buf_s.at[g2], out.at[pl.ds(off, CHUNK)], so[g2])

    def out_wait(c, g2):
        off = base + c * CHUNK
        pltpu.make_async_copy(buf_s.at[g2], out.at[pl.ds(off, CHUNK)], so[g2]).wait()

    def add_chunk(g2):
        fs = buf_f.at[g2]
        bs = buf_b.at[g2]
        ss = buf_s.at[g2]

        def add_row(i, carry):
            for j in range(D // LANES):
                sl = pl.ds(j * LANES, LANES)
                ss[i, sl] = fs[i, sl] + bs[i, sl]
            return carry

        lax.fori_loop(0, CHUNK, add_row, 0, unroll=4)

    def body_c(c, s, with_out_wait=True, with_gather=True, with_idx=True):
        g2 = s % 2
        gather_wait(s, g2)
        if with_out_wait:
            out_wait(c - 2, g2)
        add_chunk(g2)
        out_start(c, g2)
        if with_gather:
            idx_wait(c + 2, (s + 2) % 4)
            gather_start((s + 2) % 4, g2)
        if with_idx:
            idx_start(c + 4, s)

    for sl in range(4):
        idx_start(sl, sl)
    idx_wait(0, 0)
    gather_start(0, 0)
    idx_wait(1, 1)
    gather_start(1, 1)

    body_c(0, 0, with_out_wait=False)
    body_c(1, 1, with_out_wait=False)
    body_c(2, 2)
    body_c(3, 3)

    def quad(q, carry):
        c0 = q * 4
        for s in range(4):
            body_c(c0 + s, s)
        return carry

    lax.fori_loop(1, NCHUNK // 4 - 1, quad, 0)

    cl = NCHUNK - 4
    body_c(cl + 0, 0, with_idx=False)
    body_c(cl + 1, 1, with_idx=False)
    body_c(cl + 2, 2, with_gather=False, with_idx=False)
    body_c(cl + 3, 3, with_gather=False, with_idx=False)
    out_wait(NCHUNK - 2, 0)
    out_wait(NCHUNK - 1, 1)


def kernel(L_fwd, L_bwd, W_fwd, W_bwd):
    out = _encode(L_fwd.reshape(-1), L_bwd.reshape(-1), W_fwd, W_bwd)
    return out.reshape(L_fwd.shape[0], L_fwd.shape[1], D)


# speedup vs baseline: 7.6343x; 1.3602x over previous
import functools

import jax
import jax.numpy as jnp
from jax import lax
from jax.experimental import pallas as pl
from jax.experimental.pallas import tpu as pltpu
from jax.experimental.pallas import tpu_sc as plsc

NC = 2
NS = 16
LANES = 16
NW = NC * NS

ROWS = 1024 * 1000
D = 128
PER_W = ROWS // NW
CHUNK = 128
SB = 10
NCHUNK = PER_W // CHUNK
NSB = NCHUNK // SB

_mesh = plsc.VectorSubcoreMesh(core_axis_name="c", subcore_axis_name="s")


@functools.partial(
    pl.kernel,
    out_type=jax.ShapeDtypeStruct((ROWS, D), jnp.float32),
    mesh=_mesh,
    scratch_types=[
        pltpu.VMEM((2, SB * CHUNK), jnp.int32),
        pltpu.VMEM((2, SB * CHUNK), jnp.int32),
        pltpu.VMEM((2, CHUNK, D), jnp.float32),
        pltpu.VMEM((2, CHUNK, D), jnp.float32),
        pltpu.VMEM((2, CHUNK, D), jnp.float32),
        pltpu.SemaphoreType.DMA,
        pltpu.SemaphoreType.DMA,
        pltpu.SemaphoreType.DMA,
        pltpu.SemaphoreType.DMA,
        pltpu.SemaphoreType.DMA,
        pltpu.SemaphoreType.DMA,
    ],
)
def _encode(lf, lb, wf, wb, out,
            idx_f, idx_b, buf_f, buf_b, buf_s,
            si0, si1, sg0, sg1, so0, so1):
    si = (si0, si1)
    sg = (sg0, sg1)
    so = (so0, so1)
    wid = lax.axis_index("s") * NC + lax.axis_index("c")
    base = wid * PER_W

    def idx_start(sb, sl):
        off = base + sb * (SB * CHUNK)
        pltpu.async_copy(lf.at[pl.ds(off, SB * CHUNK)], idx_f.at[sl], si[sl])
        pltpu.async_copy(lb.at[pl.ds(off, SB * CHUNK)], idx_b.at[sl], si[sl])

    def idx_wait(sb, sl):
        off = base + sb * (SB * CHUNK)
        pltpu.make_async_copy(lf.at[pl.ds(off, SB * CHUNK)], idx_f.at[sl], si[sl]).wait()
        pltpu.make_async_copy(lb.at[pl.ds(off, SB * CHUNK)], idx_b.at[sl], si[sl]).wait()

    def gather_start(isl, k, g2):
        s = pl.ds(k * CHUNK, CHUNK)
        pltpu.async_copy(wf.at[idx_f.at[isl, s]], buf_f.at[g2], sg[g2])
        pltpu.async_copy(wb.at[idx_b.at[isl, s]], buf_b.at[g2], sg[g2])

    def gather_wait(isl, k, g2):
        s = pl.ds(k * CHUNK, CHUNK)
        pltpu.make_async_copy(wf.at[idx_f.at[isl, s]], buf_f.at[g2], sg[g2]).wait()
        pltpu.make_async_copy(wb.at[idx_b.at[isl, s]], buf_b.at[g2], sg[g2]).wait()

    def out_start(c, g2):
        off = base + c * CHUNK
        pltpu.async_copy(buf_s.at[g2], out.at[pl.ds(off, CHUNK)], so[g2])

    def out_wait(c, g2):
        off = base + c * CHUNK
        pltpu.make_async_copy(buf_s.at[g2], out.at[pl.ds(off, CHUNK)], so[g2]).wait()

    def add_chunk(g2):
        fs = buf_f.at[g2]
        bs = buf_b.at[g2]
        ss = buf_s.at[g2]

        def add_row(i, carry):
            for j in range(D // LANES):
                sl = pl.ds(j * LANES, LANES)
                ss[i, sl] = fs[i, sl] + bs[i, sl]
            return carry

        lax.fori_loop(0, CHUNK, add_row, 0, unroll=2)

    def body_c(sb, k, isl, out_wait_guard=None, with_gather=True,
               with_idx_wait=True, idx_start_guard=None,
               with_idx_start=True):
        c = sb * SB + k
        g2 = k % 2
        gather_wait(isl, k, g2)
        if out_wait_guard is None:
            out_wait(c - 2, g2)
        else:
            @pl.when(out_wait_guard)
            def _():
                out_wait(c - 2, g2)
        if with_gather:
            k2, isl2 = (k + 2) % SB, (isl + (1 if k + 2 >= SB else 0)) % 2
            if with_idx_wait and k == SB - 2:
                idx_wait(sb + 1, isl2)
        add_chunk(g2)
        out_start(c, g2)
        if with_gather:
            gather_start(isl2, k2, g2)
        if with_idx_start and k == SB - 1:
            if idx_start_guard is None:
                idx_start(sb + 2, isl)
            else:
                @pl.when(idx_start_guard)
                def _():
                    idx_start(sb + 2, isl)

    idx_start(0, 0)
    idx_start(1, 1)
    idx_wait(0, 0)
    gather_start(0, 0, 0)
    gather_start(0, 1, 1)

    def pair_body(i, carry):
        for k in range(SB):
            body_c(2 * i, k, 0,
                   out_wait_guard=(i > 0) if k < 2 else None)
        for k in range(SB):
            body_c(2 * i + 1, k, 1,
                   idx_start_guard=(i < NSB // 2 - 1))
        return carry

    lax.fori_loop(0, NSB // 2, pair_body, 0)

    for k in range(SB):
        body_c(NSB - 1, k, (NSB - 1) % 2, with_gather=(k < SB - 2),
               with_idx_wait=False, with_idx_start=False)
    out_wait(NCHUNK - 2, 0)
    out_wait(NCHUNK - 1, 1)


def kernel(L_fwd, L_bwd, W_fwd, W_bwd):
    out = _encode(L_fwd.reshape(-1), L_bwd.reshape(-1), W_fwd, W_bwd)
    return out.reshape(L_fwd.shape[0], L_fwd.shape[1], D)


# speedup vs baseline: 8.6588x; 1.1342x over previous
import functools

import jax
import jax.numpy as jnp
from jax import lax
from jax.experimental import pallas as pl
from jax.experimental.pallas import tpu as pltpu
from jax.experimental.pallas import tpu_sc as plsc

NC = 2
NS = 16
LANES = 16
NW = NC * NS

ROWS = 1024 * 1000
D = 128
PER_W = ROWS // NW
CHUNK = 128
SB = 10
NCHUNK = PER_W // CHUNK
NSB = NCHUNK // SB

_mesh = plsc.VectorSubcoreMesh(core_axis_name="c", subcore_axis_name="s")


@functools.partial(
    pl.kernel,
    out_type=jax.ShapeDtypeStruct((ROWS, D), jnp.float32),
    mesh=_mesh,
    scratch_types=[
        pltpu.VMEM((2, SB * CHUNK), jnp.int32),
        pltpu.VMEM((2, SB * CHUNK), jnp.int32),
        pltpu.VMEM((2, CHUNK, D), jnp.float32),
        pltpu.VMEM((2, CHUNK, D), jnp.float32),
        pltpu.VMEM((2, CHUNK, D), jnp.float32),
        pltpu.VMEM_SHARED((1000, D), jnp.float32),
        pltpu.VMEM_SHARED((1000, D), jnp.float32),
        pltpu.SemaphoreType.DMA,
        pltpu.SemaphoreType.DMA,
        pltpu.SemaphoreType.DMA,
        pltpu.SemaphoreType.DMA,
        pltpu.SemaphoreType.DMA,
        pltpu.SemaphoreType.DMA,
    ],
)
def _encode(lf, lb, wf, wb, out,
            idx_f, idx_b, buf_f, buf_b, buf_s, wf_sp, wb_sp,
            si0, si1, sg0, sg1, so0, so1):
    si = (si0, si1)
    sg = (sg0, sg1)
    so = (so0, so1)
    sid = lax.axis_index("s")
    wid = sid * NC + lax.axis_index("c")
    base = wid * PER_W

    @pl.when(sid == 0)
    def _():
        pltpu.sync_copy(wf, wf_sp)
        pltpu.sync_copy(wb, wb_sp)

    plsc.subcore_barrier()

    def idx_start(sb, sl):
        off = base + sb * (SB * CHUNK)
        pltpu.async_copy(lf.at[pl.ds(off, SB * CHUNK)], idx_f.at[sl], si[sl])
        pltpu.async_copy(lb.at[pl.ds(off, SB * CHUNK)], idx_b.at[sl], si[sl])

    def idx_wait(sb, sl):
        off = base + sb * (SB * CHUNK)
        pltpu.make_async_copy(lf.at[pl.ds(off, SB * CHUNK)], idx_f.at[sl], si[sl]).wait()
        pltpu.make_async_copy(lb.at[pl.ds(off, SB * CHUNK)], idx_b.at[sl], si[sl]).wait()

    def gather_start(isl, k, g2):
        s = pl.ds(k * CHUNK, CHUNK)
        pltpu.async_copy(wf_sp.at[idx_f.at[isl, s]], buf_f.at[g2], sg[g2])
        pltpu.async_copy(wb_sp.at[idx_b.at[isl, s]], buf_b.at[g2], sg[g2])

    def gather_wait(isl, k, g2):
        s = pl.ds(k * CHUNK, CHUNK)
        pltpu.make_async_copy(wf_sp.at[idx_f.at[isl, s]], buf_f.at[g2], sg[g2]).wait()
        pltpu.make_async_copy(wb_sp.at[idx_b.at[isl, s]], buf_b.at[g2], sg[g2]).wait()

    def out_start(c, g2):
        off = base + c * CHUNK
        pltpu.async_copy(buf_s.at[g2], out.at[pl.ds(off, CHUNK)], so[g2])

    def out_wait(c, g2):
        off = base + c * CHUNK
        pltpu.make_async_copy(buf_s.at[g2], out.at[pl.ds(off, CHUNK)], so[g2]).wait()

    def add_chunk(g2):
        fs = buf_f.at[g2]
        bs = buf_b.at[g2]
        ss = buf_s.at[g2]

        def add_row(i, carry):
            for j in range(D // LANES):
                sl = pl.ds(j * LANES, LANES)
                ss[i, sl] = fs[i, sl] + bs[i, sl]
            return carry

        lax.fori_loop(0, CHUNK, add_row, 0, unroll=2)

    def body_c(sb, k, isl, out_wait_guard=None, with_gather=True,
               with_idx_wait=True, idx_start_guard=None,
               with_idx_start=True):
        c = sb * SB + k
        g2 = k % 2
        gather_wait(isl, k, g2)
        if out_wait_guard is None:
            out_wait(c - 2, g2)
        else:
            @pl.when(out_wait_guard)
            def _():
                out_wait(c - 2, g2)
        if with_gather:
            k2, isl2 = (k + 2) % SB, (isl + (1 if k + 2 >= SB else 0)) % 2
            if with_idx_wait and k == SB - 2:
                idx_wait(sb + 1, isl2)
        add_chunk(g2)
        out_start(c, g2)
        if with_gather:
            gather_start(isl2, k2, g2)
        if with_idx_start and k == SB - 1:
            if idx_start_guard is None:
                idx_start(sb + 2, isl)
            else:
                @pl.when(idx_start_guard)
                def _():
                    idx_start(sb + 2, isl)

    idx_start(0, 0)
    idx_start(1, 1)
    idx_wait(0, 0)
    gather_start(0, 0, 0)
    gather_start(0, 1, 1)

    def pair_body(i, carry):
        for k in range(SB):
            body_c(2 * i, k, 0,
                   out_wait_guard=(i > 0) if k < 2 else None)
        for k in range(SB):
            body_c(2 * i + 1, k, 1,
                   idx_start_guard=(i < NSB // 2 - 1))
        return carry

    lax.fori_loop(0, NSB // 2, pair_body, 0)

    for k in range(SB):
        body_c(NSB - 1, k, (NSB - 1) % 2, with_gather=(k < SB - 2),
               with_idx_wait=False, with_idx_start=False)
    out_wait(NCHUNK - 2, 0)
    out_wait(NCHUNK - 1, 1)


def kernel(L_fwd, L_bwd, W_fwd, W_bwd):
    out = _encode(L_fwd.reshape(-1), L_bwd.reshape(-1), W_fwd, W_bwd)
    return out.reshape(L_fwd.shape[0], L_fwd.shape[1], D)


# speedup vs baseline: 17.3540x; 2.0042x over previous
import functools

import jax
import jax.numpy as jnp
from jax import lax
from jax.experimental import pallas as pl
from jax.experimental.pallas import tpu as pltpu
from jax.experimental.pallas import tpu_sc as plsc

NC = 2
NS = 16
NW = NC * NS

ROWS = 1024 * 1000
D = 128
PER_W = ROWS // NW
CHUNK = 128
SB = 10
NCHUNK = PER_W // CHUNK
NSB = NCHUNK // SB
NR = 5

_mesh = plsc.VectorSubcoreMesh(core_axis_name="c", subcore_axis_name="s")


@functools.partial(
    pl.kernel,
    out_type=jax.ShapeDtypeStruct((ROWS, D), jnp.float32),
    mesh=_mesh,
    scratch_types=[
        pltpu.VMEM((2, SB * CHUNK), jnp.int32),
        pltpu.VMEM((2, SB * CHUNK), jnp.int32),
        pltpu.VMEM((NR, CHUNK, D), jnp.float32),
        pltpu.VMEM_SHARED((1000, D), jnp.float32),
        pltpu.VMEM_SHARED((1000, D), jnp.float32),
        pltpu.SemaphoreType.DMA,
        pltpu.SemaphoreType.DMA,
        pltpu.SemaphoreType.DMA,
        pltpu.SemaphoreType.DMA,
        pltpu.SemaphoreType.DMA,
        pltpu.SemaphoreType.DMA,
        pltpu.SemaphoreType.DMA,
        pltpu.SemaphoreType.DMA,
        pltpu.SemaphoreType.DMA,
        pltpu.SemaphoreType.DMA,
        pltpu.SemaphoreType.DMA,
        pltpu.SemaphoreType.DMA,
        pltpu.SemaphoreType.DMA,
        pltpu.SemaphoreType.DMA,
        pltpu.SemaphoreType.DMA,
        pltpu.SemaphoreType.DMA,
        pltpu.SemaphoreType.DMA,
    ],
)
def _encode(lf, lb, wf, wb, out,
            idx_f, idx_b, buf, wf_sp, wb_sp,
            si0, si1,
            sf0, sf1, sf2, sf3, sf4,
            sb0, sb1, sb2, sb3, sb4,
            so0, so1, so2, so3, so4):
    si = (si0, si1)
    sf = (sf0, sf1, sf2, sf3, sf4)
    sbm = (sb0, sb1, sb2, sb3, sb4)
    so = (so0, so1, so2, so3, so4)
    sid = lax.axis_index("s")
    wid = sid * NC + lax.axis_index("c")
    base = wid * PER_W

    @pl.when(sid == 0)
    def _():
        pltpu.sync_copy(wf, wf_sp)
        pltpu.sync_copy(wb, wb_sp)

    plsc.subcore_barrier()

    def idx_start(sb, sl):
        off = base + sb * (SB * CHUNK)
        pltpu.async_copy(lf.at[pl.ds(off, SB * CHUNK)], idx_f.at[sl], si[sl])
        pltpu.async_copy(lb.at[pl.ds(off, SB * CHUNK)], idx_b.at[sl], si[sl])

    def idx_wait(sb, sl):
        off = base + sb * (SB * CHUNK)
        pltpu.make_async_copy(lf.at[pl.ds(off, SB * CHUNK)], idx_f.at[sl], si[sl]).wait()
        pltpu.make_async_copy(lb.at[pl.ds(off, SB * CHUNK)], idx_b.at[sl], si[sl]).wait()

    def f_start(isl, k, r):
        s = pl.ds(k * CHUNK, CHUNK)
        pltpu.async_copy(wf_sp.at[idx_f.at[isl, s]], buf.at[r], sf[r])

    def f_wait(isl, k, r):
        s = pl.ds(k * CHUNK, CHUNK)
        pltpu.make_async_copy(wf_sp.at[idx_f.at[isl, s]], buf.at[r], sf[r]).wait()

    def b_start(isl, k, r):
        s = pl.ds(k * CHUNK, CHUNK)
        pltpu.async_copy(wb_sp.at[idx_b.at[isl, s]], buf.at[r], sbm[r], add=True)

    def b_wait(isl, k, r):
        s = pl.ds(k * CHUNK, CHUNK)
        pltpu.make_async_copy(wb_sp.at[idx_b.at[isl, s]], buf.at[r], sbm[r]).wait()

    def out_start(c, r):
        off = base + c * CHUNK
        pltpu.async_copy(buf.at[r], out.at[pl.ds(off, CHUNK)], so[r])

    def out_wait(c, r):
        off = base + c * CHUNK
        pltpu.make_async_copy(buf.at[r], out.at[pl.ds(off, CHUNK)], so[r]).wait()

    def body_c(sb, k, isl, w_out=True, w_prev=True, start_f=True,
               with_idx_wait=True, with_idx_start=True):
        c = sb * SB + k
        r = k % NR
        f_wait(isl, k, r)
        b_start(isl, k, r)
        if w_prev:
            rp = (k - 1) % NR
            b_wait(isl, (k - 1) % SB, rp)
            out_start(c - 1, rp)
        if start_f:
            k2 = (k + 2) % SB
            isl2 = (isl + (1 if k + 2 >= SB else 0)) % 2
            r2 = (k + 2) % NR
            if w_out:
                out_wait(c - 3, r2)
            if with_idx_wait and k == SB - 2:
                idx_wait(sb + 1, isl2)
            f_start(isl2, k2, r2)
        if with_idx_start and k == SB - 1:
            idx_start(sb + 2, isl)

    idx_start(0, 0)
    idx_start(1, 1)
    idx_wait(0, 0)
    f_start(0, 0, 0)
    f_start(0, 1, 1)

    body_c(0, 0, 0, w_out=False, w_prev=False)
    body_c(0, 1, 0, w_out=False)
    body_c(0, 2, 0, w_out=False)
    for k in range(3, SB):
        body_c(0, k, 0)

    def pair_body(i, carry):
        sb = 1 + 2 * i
        for k in range(SB):
            body_c(sb, k, 1)
        for k in range(SB):
            body_c(sb + 1, k, 0)
        return carry

    lax.fori_loop(0, (NSB - 3) // 2, pair_body, 0)

    for k in range(SB):
        body_c(NSB - 2, k, (NSB - 2) % 2, with_idx_start=False)

    lsl = (NSB - 1) % 2
    for k in range(SB):
        body_c(NSB - 1, k, lsl, start_f=(k < SB - 2),
               with_idx_wait=False, with_idx_start=False)

    last = NCHUNK - 1
    rl = last % NR
    b_wait(lsl, (SB - 1) % SB, rl)
    out_start(last, rl)
    for c in range(NCHUNK - NR, NCHUNK):
        out_wait(c, c % NR)


def kernel(L_fwd, L_bwd, W_fwd, W_bwd):
    out = _encode(L_fwd.reshape(-1), L_bwd.reshape(-1), W_fwd, W_bwd)
    return out.reshape(L_fwd.shape[0], L_fwd.shape[1], D)
